# baseline (speedup 1.0000x reference)
import functools

import jax
import jax.numpy as jnp
from jax import lax
from jax.experimental import pallas as pl
from jax.experimental.pallas import tpu as pltpu
from jax.experimental.pallas import tpu_sc as plsc

_NOISE = 25_000_000
_P = 102928
_N = 256
_CLIP = 40.0

_NW = 32
_C = 3232
_PTOT = _NW * _C
_W = 3248
_WBUF = 3760
_NCH = _C // 16
_NCH_LAST = (_P - 31 * _C) // 16


def _weights_body(acol, arow, bcol, brow, scol, sign_ref, coef_out, c0_out):
    a_c = -acol[...]
    a_r = -arow[...]
    b_c = -bcol[...]
    b_r = -brow[...]
    ii = lax.broadcasted_iota(jnp.int32, (_N, _N), 0)
    jj = lax.broadcasted_iota(jnp.int32, (_N, _N), 1)

    def cnt(xr, xc, tie):
        lt = (xr < xc).astype(jnp.float32)
        eq = ((xr == xc) & tie).astype(jnp.float32)
        return jnp.sum(lt + eq, axis=1, keepdims=True)

    rank_a = cnt(a_r, a_c, jj < ii) + cnt(b_r, a_c, jj < ii)
    rank_b = cnt(a_r, b_c, jj <= ii) + cnt(b_r, b_c, jj < ii)
    w = (rank_a - rank_b) * jnp.float32(1.0 / 511.0)
    coef_out[...] = w * scol[...] * jnp.float32(1.0 / (2.0 * _N))
    c0_out[...] = sign_ref[...] * scol[0:1, :]


def _clip_body(dsc, dtc, delta_out):
    d = dsc[pl.ds(0, _P)] + dtc[...]
    total = jnp.sum(d * d)
    gnorm = jnp.sqrt(total)
    factor = _CLIP / jnp.maximum(gnorm, _CLIP)
    delta_out[...] = d * factor


_G = 8
_K = 2 * _G

_NT = 96
_NTO = 64
_NSC = _N - _NT
_W2E = 103168
_ENDOFF = 24896000
_ENDLEN = 104192


def _seed_slot(m):
    return jnp.where(m < _NTO, m, m + _NT) if not isinstance(m, int) else (
        m if m < _NTO else m + _NT)


def _tc_accum_body(noise_hbm, end_hbm, seeds_smem, coef_smem, out_ref,
                   acc, buf0, buf1, sem0, sem1):
    nrow = _PTOT // 128
    lane = lax.broadcasted_iota(jnp.int32, (nrow, 128), 1)
    acc[...] = jnp.zeros((nrow, 128), jnp.float32)
    ztail = jnp.zeros((_PTOT - _W2E,), jnp.float32)
    buf0[pl.ds(_W2E, _PTOT - _W2E)] = ztail
    buf1[pl.ds(_W2E, _PTOT - _W2E)] = ztail

    def dmap(i):
        o = seeds_smem[i]
        oa = pl.multiple_of((o >> 7) << 7, 128)
        return oa, o - oa

    def start(oa, buf, sem):
        @pl.when(oa <= _NOISE - _W2E)
        def _():
            pltpu.make_async_copy(
                noise_hbm.at[pl.ds(oa, _W2E)],
                buf.at[pl.ds(0, _W2E)], sem).start()

        @pl.when(oa > _NOISE - _W2E)
        def _():
            oe = pl.multiple_of(oa - _ENDOFF, 128)
            pltpu.make_async_copy(
                end_hbm.at[pl.ds(oe, _W2E)],
                buf.at[pl.ds(0, _W2E)], sem).start()

    def wait(buf, sem):
        pltpu.make_async_copy(
            noise_hbm.at[pl.ds(0, _W2E)], buf.at[pl.ds(0, _W2E)], sem).wait()

    def process(i, buf, r):
        b2 = jnp.reshape(buf[...], (nrow, 128))
        rot_same = pltpu.roll(b2, 128 - r, 1)
        rot_next = pltpu.roll(rot_same, nrow - 1, 0)
        win = jnp.where(lane < 128 - r, rot_same, rot_next)
        acc[...] = acc[...] + win * coef_smem[i]

    oa_a, r_a = dmap(_NTO)
    start(oa_a, buf0, sem0)

    def body(g, r0):
        i0 = _NTO + 2 * g
        oa_b, r1 = dmap(i0 + 1)
        start(oa_b, buf1, sem1)
        wait(buf0, sem0)
        process(i0, buf0, r0)
        oa_n, rn = dmap(i0 + 2)

        @pl.when(2 * g + 2 < _NT)
        def _():
            start(oa_n, buf0, sem0)

        wait(buf1, sem1)
        process(i0 + 1, buf1, r1)
        return rn

    lax.fori_loop(0, _NT // 2, body, r_a)
    flat = jnp.reshape(acc[...], (_PTOT,))
    out_ref[...] = lax.slice(flat, (0,), (_P,))


_tc_accum = pl.pallas_call(
    _tc_accum_body,
    out_shape=jax.ShapeDtypeStruct((_P,), jnp.float32),
    in_specs=[
        pl.BlockSpec(memory_space=pltpu.MemorySpace.HBM),
        pl.BlockSpec(memory_space=pltpu.MemorySpace.HBM),
        pl.BlockSpec(memory_space=pltpu.SMEM),
        pl.BlockSpec(memory_space=pltpu.SMEM),
    ],
    scratch_shapes=[
        pltpu.VMEM((_PTOT // 128, 128), jnp.float32),
        pltpu.VMEM((_PTOT,), jnp.float32),
        pltpu.VMEM((_PTOT,), jnp.float32),
        pltpu.SemaphoreType.DMA,
        pltpu.SemaphoreType.DMA,
    ],
)


def _sc_body(noise_hbm, seeds_hbm, coef_hbm, params_hbm,
             delta_hbm, pert_hbm,
             seeds_v, coef_v, acc, pv, *rest):
    bufs = rest[:_K]
    sems = rest[_K:]
    wid = lax.axis_index("s") * 2 + lax.axis_index("c")
    s_base = pl.multiple_of(wid * _C, 8)
    lanes = lax.iota(jnp.int32, 16)

    pltpu.sync_copy(seeds_hbm, seeds_v)
    pltpu.sync_copy(coef_hbm, coef_v)
    _CL = _NCH_LAST * 16

    @pl.when(wid != _NW - 1)
    def _():
        pltpu.sync_copy(params_hbm.at[pl.ds(s_base, _C)], pv)

    @pl.when(wid == _NW - 1)
    def _():
        pltpu.sync_copy(params_hbm.at[pl.ds(s_base, _CL)], pv.at[pl.ds(0, _CL)])

    def extract(ref, i):
        return ref[pl.ds(i, 16)][0]

    zeros16 = jnp.zeros((16,), jnp.float32)

    @plsc.parallel_loop(0, _NCH)
    def _(j):
        acc[pl.ds(j * 16, 16)] = zeros16

    @plsc.parallel_loop(0, (_WBUF - _W) // 16)
    def _(j):
        for _b in range(_K):
            bufs[_b][pl.ds(_W + j * 16, 16)] = zeros16

    def dma_params(i):
        o = extract(seeds_v, i) + s_base
        oa = pl.multiple_of(jnp.minimum((o >> 3) << 3, _NOISE - _W), 8)
        return oa, o - oa

    def start(oa, buf, sem):
        pltpu.make_async_copy(
            noise_hbm.at[pl.ds(oa, _W)], buf.at[pl.ds(0, _W)], sem).start()

    def wait(buf, sem):
        pltpu.make_async_copy(
            noise_hbm.at[pl.ds(0, _W)], buf.at[pl.ds(0, _W)], sem).wait()

    def accumulate8(grp, rs8, cs8):
        @plsc.parallel_loop(0, _NCH, unroll=4)
        def _(j):
            t = cs8[0] * grp[0][pl.ds(rs8[0] + j * 16, 16)]
            for b in range(1, _G):
                t = t + cs8[b] * grp[b][pl.ds(rs8[b] + j * 16, 16)]
            plsc.addupdate(acc.at[pl.ds(j * 16, 16)], t)

    rs0 = []
    for b in range(_K):
        oa_b, r_b = dma_params(b)
        start(oa_b, bufs[b], sems[b])
        rs0.append(r_b)

    def half(h, rs, lo):
        rs = list(rs)
        grp = bufs[lo:lo + _G]
        for b in range(_G):
            wait(grp[b], sems[lo + b])
        cs8 = [extract(coef_v, _seed_slot(h * _K + lo + b)) for b in range(_G)]
        accumulate8(grp, rs[lo:lo + _G], cs8)

        if lo == 0:
            @pl.when(h == 0)
            def _():
                c0 = extract(coef_v, _N)
                r0 = rs[0]

                @plsc.parallel_loop(0, _NCH, unroll=8)
                def _(j):
                    v = bufs[0][pl.ds(r0 + j * 16, 16)]
                    plsc.addupdate(pv.at[pl.ds(j * 16, 16)], c0 * v)

        for b in range(_G):
            inext = h * _K + lo + b + _K
            oan, rn = dma_params(_seed_slot(inext))

            @pl.when(inext < _NSC)
            def _():
                start(oan, bufs[lo + b], sems[lo + b])

            rs[lo + b] = rn
        return rs

    def body(h, rs):
        rs = half(h, rs, 0)
        rs = half(h, rs, _G)
        return tuple(rs)

    lax.fori_loop(0, _NSC // _K, body, tuple(rs0))

    pltpu.sync_copy(acc, delta_hbm.at[pl.ds(s_base, _C)])

    @pl.when(wid != _NW - 1)
    def _():
        pltpu.sync_copy(pv, pert_hbm.at[pl.ds(s_base, _C)])

    @pl.when(wid == _NW - 1)
    def _():
        pltpu.sync_copy(pv.at[pl.ds(0, _CL)], pert_hbm.at[pl.ds(s_base, _CL)])


@functools.partial(
    pl.kernel,
    out_type=(
        jax.ShapeDtypeStruct((_PTOT,), jnp.float32),
        jax.ShapeDtypeStruct((_P,), jnp.float32),
    ),
    mesh=plsc.VectorSubcoreMesh(core_axis_name="c", subcore_axis_name="s"),
    scratch_types=(
        pltpu.VMEM((_N + 32,), jnp.int32),
        pltpu.VMEM((_N + 32,), jnp.float32),
        pltpu.VMEM((_C,), jnp.float32),
        pltpu.VMEM((_C,), jnp.float32),
    ) + tuple(pltpu.VMEM((_WBUF,), jnp.float32) for _ in range(_K))
      + tuple(pltpu.SemaphoreType.DMA for _ in range(_K)),
    compiler_params=pltpu.CompilerParams(needs_layout_passes=False),
)
def _sc_call(noise_hbm, seeds_hbm, coef_hbm, params_hbm,
             delta_hbm, pert_hbm, *rest):
    _sc_body(noise_hbm, seeds_hbm, coef_hbm, params_hbm,
             delta_hbm, pert_hbm, *rest)


def kernel(noise_table, params, perturbation_seeds, returns,
           perturbation_scales, positive_perturbation):
    f32 = jnp.float32
    acol = returns[:, 0].reshape(_N, 1)
    arow = returns[:, 0].reshape(1, _N)
    bcol = returns[:, 1].reshape(_N, 1)
    brow = returns[:, 1].reshape(1, _N)
    scol = perturbation_scales.reshape(_N, 1)
    sgn = (2.0 * jnp.asarray(positive_perturbation, f32) - 1.0).reshape(1, 1)

    coef, c0 = pl.pallas_call(
        _weights_body,
        out_shape=(
            jax.ShapeDtypeStruct((_N, 1), f32),
            jax.ShapeDtypeStruct((1, 1), f32),
        ),
    )(acol, arow, bcol, brow, scol, sgn)

    coef_ext = jnp.concatenate(
        [coef.reshape(-1), c0.reshape(-1), jnp.zeros(31, f32)])
    seeds_ext = jnp.concatenate(
        [perturbation_seeds.astype(jnp.int32), jnp.zeros(32, jnp.int32)])
    noise_end = jnp.concatenate(
        [noise_table[_ENDOFF:], jnp.zeros(_ENDLEN - (_NOISE - _ENDOFF), f32)])
    dtc = _tc_accum(noise_table, noise_end, seeds_ext, coef_ext)
    delta_raw, pert = _sc_call(
        noise_table, seeds_ext, coef_ext, params)

    delta = pl.pallas_call(
        _clip_body,
        out_shape=jax.ShapeDtypeStruct((_P,), f32),
    )(delta_raw, dtc)

    return delta, pert


# speedup vs baseline: 1.2608x; 1.2608x over previous
import functools

import jax
import jax.numpy as jnp
from jax import lax
from jax.experimental import pallas as pl
from jax.experimental.pallas import tpu as pltpu
from jax.experimental.pallas import tpu_sc as plsc

_NOISE = 25_000_000
_P = 102928
_N = 256
_CLIP = 40.0

_NW = 32
_C = 3232
_PTOT = _NW * _C
_W = 3248
_WBUF = 3760
_NCH = _C // 16
_NCH_LAST = (_P - 31 * _C) // 16


def _weights_body(acol, arow, bcol, brow, scol, sign_ref, coef_out, c0_out):
    a_c = -acol[...]
    a_r = -arow[...]
    b_c = -bcol[...]
    b_r = -brow[...]
    ii = lax.broadcasted_iota(jnp.int32, (_N, _N), 0)
    jj = lax.broadcasted_iota(jnp.int32, (_N, _N), 1)

    def cnt(xr, xc, tie):
        lt = (xr < xc).astype(jnp.float32)
        eq = ((xr == xc) & tie).astype(jnp.float32)
        return jnp.sum(lt + eq, axis=1, keepdims=True)

    rank_a = cnt(a_r, a_c, jj < ii) + cnt(b_r, a_c, jj < ii)
    rank_b = cnt(a_r, b_c, jj <= ii) + cnt(b_r, b_c, jj < ii)
    w = (rank_a - rank_b) * jnp.float32(1.0 / 511.0)
    coef_out[...] = w * scol[...] * jnp.float32(1.0 / (2.0 * _N))
    c0_out[...] = sign_ref[...] * scol[0:1, :]


def _clip_body(dsc, dtc, delta_out):
    d = dsc[pl.ds(0, _P)] + dtc[...]
    total = jnp.sum(d * d)
    gnorm = jnp.sqrt(total)
    factor = _CLIP / jnp.maximum(gnorm, _CLIP)
    delta_out[...] = d * factor


_G = 8
_K = 2 * _G

_NT = 48
_NTO = 64
_NSC = _N - _NT
_W2E = 103168
_ENDOFF = 24896000
_ENDLEN = 104192


def _seed_slot(m):
    return jnp.where(m < _NTO, m, m + _NT) if not isinstance(m, int) else (
        m if m < _NTO else m + _NT)


def _tc_accum_body(noise_hbm, end_hbm, seeds_smem, coef_smem, out_ref,
                   acc, buf0, buf1, sem0, sem1):
    nrow = _PTOT // 128
    lane = lax.broadcasted_iota(jnp.int32, (nrow, 128), 1)
    acc[...] = jnp.zeros((nrow, 128), jnp.float32)
    ztail = jnp.zeros((_PTOT - _W2E,), jnp.float32)
    buf0[pl.ds(_W2E, _PTOT - _W2E)] = ztail
    buf1[pl.ds(_W2E, _PTOT - _W2E)] = ztail

    def dmap(i):
        o = seeds_smem[i]
        oa = pl.multiple_of((o >> 7) << 7, 128)
        return oa, o - oa

    def start(oa, buf, sem):
        @pl.when(oa <= _NOISE - _W2E)
        def _():
            pltpu.make_async_copy(
                noise_hbm.at[pl.ds(oa, _W2E)],
                buf.at[pl.ds(0, _W2E)], sem).start()

        @pl.when(oa > _NOISE - _W2E)
        def _():
            oe = pl.multiple_of(oa - _ENDOFF, 128)
            pltpu.make_async_copy(
                end_hbm.at[pl.ds(oe, _W2E)],
                buf.at[pl.ds(0, _W2E)], sem).start()

    def wait(buf, sem):
        pltpu.make_async_copy(
            noise_hbm.at[pl.ds(0, _W2E)], buf.at[pl.ds(0, _W2E)], sem).wait()

    def process(i, buf, r):
        b2 = jnp.reshape(buf[...], (nrow, 128))
        rot_same = pltpu.roll(b2, 128 - r, 1)
        rot_next = pltpu.roll(rot_same, nrow - 1, 0)
        win = jnp.where(lane < 128 - r, rot_same, rot_next)
        acc[...] = acc[...] + win * coef_smem[i]

    oa_a, r_a = dmap(_NTO)
    start(oa_a, buf0, sem0)

    def body(g, r0):
        i0 = _NTO + 2 * g
        oa_b, r1 = dmap(i0 + 1)
        start(oa_b, buf1, sem1)
        wait(buf0, sem0)
        process(i0, buf0, r0)
        oa_n, rn = dmap(i0 + 2)

        @pl.when(2 * g + 2 < _NT)
        def _():
            start(oa_n, buf0, sem0)

        wait(buf1, sem1)
        process(i0 + 1, buf1, r1)
        return rn

    lax.fori_loop(0, _NT // 2, body, r_a)
    flat = jnp.reshape(acc[...], (_PTOT,))
    out_ref[...] = lax.slice(flat, (0,), (_P,))


_tc_accum = pl.pallas_call(
    _tc_accum_body,
    out_shape=jax.ShapeDtypeStruct((_P,), jnp.float32),
    in_specs=[
        pl.BlockSpec(memory_space=pltpu.MemorySpace.HBM),
        pl.BlockSpec(memory_space=pltpu.MemorySpace.HBM),
        pl.BlockSpec(memory_space=pltpu.SMEM),
        pl.BlockSpec(memory_space=pltpu.SMEM),
    ],
    scratch_shapes=[
        pltpu.VMEM((_PTOT // 128, 128), jnp.float32),
        pltpu.VMEM((_PTOT,), jnp.float32),
        pltpu.VMEM((_PTOT,), jnp.float32),
        pltpu.SemaphoreType.DMA,
        pltpu.SemaphoreType.DMA,
    ],
)


def _sc_body(noise_hbm, seeds_hbm, coef_hbm, params_hbm,
             delta_hbm, pert_hbm,
             seeds_v, coef_v, acc, pv, *rest):
    bufs = rest[:_K]
    sems = rest[_K:]
    wid = lax.axis_index("s") * 2 + lax.axis_index("c")
    s_base = pl.multiple_of(wid * _C, 8)
    lanes = lax.iota(jnp.int32, 16)

    pltpu.sync_copy(seeds_hbm, seeds_v)
    pltpu.sync_copy(coef_hbm, coef_v)
    _CL = _NCH_LAST * 16

    @pl.when(wid != _NW - 1)
    def _():
        pltpu.sync_copy(params_hbm.at[pl.ds(s_base, _C)], pv)

    @pl.when(wid == _NW - 1)
    def _():
        pltpu.sync_copy(params_hbm.at[pl.ds(s_base, _CL)], pv.at[pl.ds(0, _CL)])

    def extract(ref, i):
        return ref[pl.ds(i, 16)][0]

    zeros16 = jnp.zeros((16,), jnp.float32)

    @plsc.parallel_loop(0, _NCH)
    def _(j):
        acc[pl.ds(j * 16, 16)] = zeros16

    @plsc.parallel_loop(0, (_WBUF - _W) // 16)
    def _(j):
        for _b in range(_K):
            bufs[_b][pl.ds(_W + j * 16, 16)] = zeros16

    def dma_params(i):
        o = extract(seeds_v, i) + s_base
        oa = pl.multiple_of(jnp.minimum((o >> 3) << 3, _NOISE - _W), 8)
        return oa, o - oa

    def start(oa, buf, sem):
        pltpu.make_async_copy(
            noise_hbm.at[pl.ds(oa, _W)], buf.at[pl.ds(0, _W)], sem).start()

    def wait(buf, sem):
        pltpu.make_async_copy(
            noise_hbm.at[pl.ds(0, _W)], buf.at[pl.ds(0, _W)], sem).wait()

    def accumulate8(grp, rs8, cs8):
        @plsc.parallel_loop(0, _NCH, unroll=4)
        def _(j):
            t = cs8[0] * grp[0][pl.ds(rs8[0] + j * 16, 16)]
            for b in range(1, _G):
                t = t + cs8[b] * grp[b][pl.ds(rs8[b] + j * 16, 16)]
            plsc.addupdate(acc.at[pl.ds(j * 16, 16)], t)

    rs0 = []
    for b in range(_K):
        oa_b, r_b = dma_params(b)
        start(oa_b, bufs[b], sems[b])
        rs0.append(r_b)

    def half(h, rs, lo):
        rs = list(rs)
        grp = bufs[lo:lo + _G]
        for b in range(_G):
            wait(grp[b], sems[lo + b])
        cs8 = [extract(coef_v, _seed_slot(h * _K + lo + b)) for b in range(_G)]
        accumulate8(grp, rs[lo:lo + _G], cs8)

        if lo == 0:
            @pl.when(h == 0)
            def _():
                c0 = extract(coef_v, _N)
                r0 = rs[0]

                @plsc.parallel_loop(0, _NCH, unroll=8)
                def _(j):
                    v = bufs[0][pl.ds(r0 + j * 16, 16)]
                    plsc.addupdate(pv.at[pl.ds(j * 16, 16)], c0 * v)

        for b in range(_G):
            inext = h * _K + lo + b + _K
            oan, rn = dma_params(_seed_slot(inext))

            @pl.when(inext < _NSC)
            def _():
                start(oan, bufs[lo + b], sems[lo + b])

            rs[lo + b] = rn
        return rs

    def body(h, rs):
        rs = half(h, rs, 0)
        rs = half(h, rs, _G)
        return tuple(rs)

    lax.fori_loop(0, _NSC // _K, body, tuple(rs0))

    pltpu.sync_copy(acc, delta_hbm.at[pl.ds(s_base, _C)])

    @pl.when(wid != _NW - 1)
    def _():
        pltpu.sync_copy(pv, pert_hbm.at[pl.ds(s_base, _C)])

    @pl.when(wid == _NW - 1)
    def _():
        pltpu.sync_copy(pv.at[pl.ds(0, _CL)], pert_hbm.at[pl.ds(s_base, _CL)])


@functools.partial(
    pl.kernel,
    out_type=(
        jax.ShapeDtypeStruct((_PTOT,), jnp.float32),
        jax.ShapeDtypeStruct((_P,), jnp.float32),
    ),
    mesh=plsc.VectorSubcoreMesh(core_axis_name="c", subcore_axis_name="s"),
    scratch_types=(
        pltpu.VMEM((_N + 32,), jnp.int32),
        pltpu.VMEM((_N + 32,), jnp.float32),
        pltpu.VMEM((_C,), jnp.float32),
        pltpu.VMEM((_C,), jnp.float32),
    ) + tuple(pltpu.VMEM((_WBUF,), jnp.float32) for _ in range(_K))
      + tuple(pltpu.SemaphoreType.DMA for _ in range(_K)),
    compiler_params=pltpu.CompilerParams(needs_layout_passes=False),
)
def _sc_call(noise_hbm, seeds_hbm, coef_hbm, params_hbm,
             delta_hbm, pert_hbm, *rest):
    _sc_body(noise_hbm, seeds_hbm, coef_hbm, params_hbm,
             delta_hbm, pert_hbm, *rest)


def kernel(noise_table, params, perturbation_seeds, returns,
           perturbation_scales, positive_perturbation):
    f32 = jnp.float32
    acol = returns[:, 0].reshape(_N, 1)
    arow = returns[:, 0].reshape(1, _N)
    bcol = returns[:, 1].reshape(_N, 1)
    brow = returns[:, 1].reshape(1, _N)
    scol = perturbation_scales.reshape(_N, 1)
    sgn = (2.0 * jnp.asarray(positive_perturbation, f32) - 1.0).reshape(1, 1)

    coef, c0 = pl.pallas_call(
        _weights_body,
        out_shape=(
            jax.ShapeDtypeStruct((_N, 1), f32),
            jax.ShapeDtypeStruct((1, 1), f32),
        ),
    )(acol, arow, bcol, brow, scol, sgn)

    coef_ext = jnp.concatenate(
        [coef.reshape(-1), c0.reshape(-1), jnp.zeros(31, f32)])
    seeds_ext = jnp.concatenate(
        [perturbation_seeds.astype(jnp.int32), jnp.zeros(32, jnp.int32)])
    noise_end = jnp.concatenate(
        [noise_table[_ENDOFF:], jnp.zeros(_ENDLEN - (_NOISE - _ENDOFF), f32)])
    dtc = _tc_accum(noise_table, noise_end, seeds_ext, coef_ext)
    delta_raw, pert = _sc_call(
        noise_table, seeds_ext, coef_ext, params)

    delta = pl.pallas_call(
        _clip_body,
        out_shape=jax.ShapeDtypeStruct((_P,), f32),
    )(delta_raw, dtc)

    return delta, pert
